# baseline (speedup 1.0000x reference)
import jax
import jax.numpy as jnp
from jax.experimental import pallas as pl

_B = 2000
_MAX_ID = 189000
_STRIDE = 3000


def _copy_kernel(x_ref, o_ref):
    i = pl.program_id(0)
    j = pl.program_id(1)
    row = j * _B + jax.lax.broadcasted_iota(jnp.int32, (_B, 128), 0)
    x = x_ref[0]
    drop = (i == 0) & (row % _STRIDE == 0) & (row <= _MAX_ID)
    o_ref[0] = jnp.where(drop, 0.0, x)


def kernel(data):
    n = data.shape[1]
    return pl.pallas_call(
        _copy_kernel,
        grid=(data.shape[0], n // _B),
        in_specs=[pl.BlockSpec((1, _B, 128), lambda i, j: (i, j, 0))],
        out_specs=pl.BlockSpec((1, _B, 128), lambda i, j: (i, j, 0)),
        out_shape=jax.ShapeDtypeStruct(data.shape, data.dtype),
    )(data)


# speedup vs baseline: 1.4921x; 1.4921x over previous
import jax
import jax.numpy as jnp
from jax.experimental import pallas as pl

_B = 8000
_MAX_ID = 189000
_STRIDE = 3000


def _copy_kernel(x_ref, o_ref):
    i = pl.program_id(0)
    j = pl.program_id(1)
    row = j * _B + jax.lax.broadcasted_iota(jnp.int32, (_B, 128), 0)
    x = x_ref[0]
    drop = (i == 0) & (row % _STRIDE == 0) & (row <= _MAX_ID)
    o_ref[0] = jnp.where(drop, 0.0, x)


def kernel(data):
    n = data.shape[1]
    return pl.pallas_call(
        _copy_kernel,
        grid=(data.shape[0], n // _B),
        in_specs=[pl.BlockSpec((1, _B, 128), lambda i, j: (i, j, 0))],
        out_specs=pl.BlockSpec((1, _B, 128), lambda i, j: (i, j, 0)),
        out_shape=jax.ShapeDtypeStruct(data.shape, data.dtype),
    )(data)


# speedup vs baseline: 1.6542x; 1.1086x over previous
import jax
import jax.numpy as jnp
from jax.experimental import pallas as pl

_B = 20000
_MAX_ID = 189000
_STRIDE = 3000


def _copy_kernel(x_ref, o_ref):
    i = pl.program_id(0)
    j = pl.program_id(1)
    row = j * _B + jax.lax.broadcasted_iota(jnp.int32, (_B, 128), 0)
    x = x_ref[0]
    drop = (i == 0) & (row % _STRIDE == 0) & (row <= _MAX_ID)
    o_ref[0] = jnp.where(drop, 0.0, x)


def kernel(data):
    n = data.shape[1]
    return pl.pallas_call(
        _copy_kernel,
        grid=(data.shape[0], n // _B),
        in_specs=[pl.BlockSpec((1, _B, 128), lambda i, j: (i, j, 0))],
        out_specs=pl.BlockSpec((1, _B, 128), lambda i, j: (i, j, 0)),
        out_shape=jax.ShapeDtypeStruct(data.shape, data.dtype),
    )(data)
